# baseline (device time: 48575 ns/iter reference)
import jax
import jax.numpy as jnp
from jax import lax
from jax.experimental import pallas as pl
from jax.experimental.pallas import tpu as pltpu

N_DEV = 8
N_GROUPS = 6
ORDERS = tuple(((0, 1, 2), (1, 2, 0), (2, 0, 1))[g % 3] for g in range(N_GROUPS))


def kernel(A, B):
    m, k = A.shape
    k2, n = B.shape
    assert k == k2
    g_rows = m // N_GROUPS
    h_rows = g_rows // 2
    q_rows = g_rows // 4

    def body(a_hbm, b_hbm, out_hbm,
             av, bv, acc, mir_ref,
             rs0, rs1, rs2,
             send_sems, recv_sems,
             a_sems, b_sem, out_sems):
        p = lax.axis_index("i")
        plane = lax.rem(p, 4)
        zc = lax.div(p, 4)
        bx = jnp.where((plane == 1) | (plane == 2), 1, 0)
        by = jnp.where(plane >= 2, 1, 0)
        bz = zc
        nx = jnp.bitwise_xor(p, 1)
        ny = 4 * zc + (3 - plane)
        nz = jnp.bitwise_xor(p, 4)
        ax = ((nx, bx), (ny, by), (nz, bz))

        b_copy = pltpu.make_async_copy(b_hbm, bv, b_sem)
        b_copy.start()
        rb = [g_rows * g for g in range(N_GROUPS)]
        sb0 = [None] * N_GROUPS
        a_copies = [[None] * N_GROUPS, [None] * N_GROUPS]
        for g in range(N_GROUPS):
            _, bit = ax[ORDERS[g][0]]
            sb0[g] = rb[g] + (1 - bit) * h_rows
            rb[g] = rb[g] + bit * h_rows
        for half, base in ((0, sb0), (1, rb)):
            for g in range(N_GROUPS):
                a_copies[half][g] = pltpu.make_async_copy(
                    a_hbm.at[pl.ds(base[g], h_rows), :],
                    av.at[pl.ds(base[g], h_rows), :],
                    a_sems.at[half, g],
                )
                a_copies[half][g].start()

        barrier_sem = pltpu.get_barrier_semaphore()
        for nbr, _ in ax:
            pl.semaphore_signal(
                barrier_sem, inc=1,
                device_id=(nbr,), device_id_type=pl.DeviceIdType.MESH,
            )

        def exchange(g, step, axis, src_base, blk, dst_ref):
            nbr, _ = ax[axis]
            rdma = pltpu.make_async_remote_copy(
                src_ref=mir_ref.at[pl.ds(src_base, blk), :],
                dst_ref=dst_ref,
                send_sem=send_sems.at[step, g],
                recv_sem=recv_sems.at[step, g],
                device_id=(nbr,),
                device_id_type=pl.DeviceIdType.MESH,
            )
            rdma.start()
            return rdma

        out_copies = []

        def store_out(idx, g, base, rows):
            cp = pltpu.make_async_copy(
                acc.at[pl.ds(base, rows), :],
                out_hbm.at[pl.ds(base, rows), :],
                out_sems.at[idx, g],
            )
            cp.start()
            out_copies.append(cp)

        rdmas = [None] * N_GROUPS
        b_copy.wait()
        for g in range(N_GROUPS):
            a1 = ORDERS[g][0]
            a_copies[0][g].wait()
            mir_ref[pl.ds(sb0[g], h_rows), :] = jnp.dot(
                av[pl.ds(sb0[g], h_rows), :], bv[:, :],
                preferred_element_type=jnp.float32,
            ).astype(jnp.bfloat16)
            if g == 0:
                pl.semaphore_wait(barrier_sem, 3)
            rdmas[g] = exchange(g, 0, a1, sb0[g], h_rows, rs0.at[g])
        for g in range(N_GROUPS):
            a_copies[1][g].wait()
            acc[pl.ds(rb[g], h_rows), :] = jnp.dot(
                av[pl.ds(rb[g], h_rows), :], bv[:, :],
                preferred_element_type=jnp.float32,
            )

        for g in range(N_GROUPS):
            a2 = ORDERS[g][1]
            _, bit = ax[a2]
            rdmas[g].wait()
            soff = (1 - bit) * q_rows
            sb = rb[g] + soff
            kb = rb[g] + (q_rows - soff)
            mir_ref[pl.ds(sb, q_rows), :] = (
                acc[pl.ds(sb, q_rows), :]
                + rs0[g, pl.ds(soff, q_rows), :].astype(jnp.float32)
            ).astype(jnp.bfloat16)
            rdmas[g] = exchange(g, 1, a2, sb, q_rows, rs1.at[g])
            acc[pl.ds(kb, q_rows), :] += rs0[
                g, pl.ds(q_rows - soff, q_rows), :
            ].astype(jnp.float32)
            rb[g] = rb[g] + bit * q_rows

        for g in range(N_GROUPS):
            a3 = ORDERS[g][2]
            rdmas[g].wait()
            s = acc[pl.ds(rb[g], q_rows), :] + rs1[g, :, :].astype(jnp.float32)
            acc[pl.ds(rb[g], q_rows), :] = s
            mir_ref[pl.ds(rb[g], q_rows), :] = s.astype(jnp.bfloat16)
            rdmas[g] = exchange(g, 2, a3, rb[g], q_rows, rs2.at[g])

        pvb = [None] * N_GROUPS
        for g in range(N_GROUPS):
            a2 = ORDERS[g][1]
            _, bit = ax[a2]
            rdmas[g].wait()
            s = acc[pl.ds(rb[g], q_rows), :] + rs2[g, :, :].astype(jnp.float32)
            acc[pl.ds(rb[g], q_rows), :] = s
            mir_ref[pl.ds(rb[g], q_rows), :] = s.astype(jnp.bfloat16)
            rdmas[g] = exchange(
                g, 3, a2, rb[g], q_rows,
                mir_ref.at[pl.ds(rb[g], q_rows), :],
            )
            store_out(0, g, rb[g], q_rows)
            pvb[g] = rb[g] + (1 - 2 * bit) * q_rows
            rb[g] = rb[g] - bit * q_rows

        pvb2 = [None] * N_GROUPS
        for g in range(N_GROUPS):
            a1 = ORDERS[g][0]
            _, bit = ax[a1]
            rdmas[g].wait()
            rdmas[g] = exchange(
                g, 4, a1, rb[g], h_rows,
                mir_ref.at[pl.ds(rb[g], h_rows), :],
            )
            pvb2[g] = rb[g] + (1 - 2 * bit) * h_rows
            acc[pl.ds(pvb[g], q_rows), :] = mir_ref[
                pl.ds(pvb[g], q_rows), :
            ].astype(jnp.float32)
            store_out(1, g, pvb[g], q_rows)

        for g in range(N_GROUPS):
            rdmas[g].wait()
            acc[pl.ds(pvb2[g], h_rows), :] = mir_ref[
                pl.ds(pvb2[g], h_rows), :
            ].astype(jnp.float32)
            store_out(2, g, pvb2[g], h_rows)

        for cp in out_copies:
            cp.wait()

    return pl.pallas_call(
        body,
        out_shape=jax.ShapeDtypeStruct((m, n), jnp.float32),
        in_specs=[
            pl.BlockSpec(memory_space=pl.ANY),
            pl.BlockSpec(memory_space=pl.ANY),
        ],
        out_specs=pl.BlockSpec(memory_space=pl.ANY),
        scratch_shapes=[
            pltpu.VMEM((m, k), jnp.float32),
            pltpu.VMEM((k, n), jnp.float32),
            pltpu.VMEM((m, n), jnp.float32),
            pltpu.VMEM((m, n), jnp.bfloat16),
            pltpu.VMEM((N_GROUPS, h_rows, n), jnp.bfloat16),
            pltpu.VMEM((N_GROUPS, q_rows, n), jnp.bfloat16),
            pltpu.VMEM((N_GROUPS, q_rows, n), jnp.bfloat16),
            pltpu.SemaphoreType.DMA((5, N_GROUPS)),
            pltpu.SemaphoreType.DMA((5, N_GROUPS)),
            pltpu.SemaphoreType.DMA((2, N_GROUPS)),
            pltpu.SemaphoreType.DMA,
            pltpu.SemaphoreType.DMA((3, N_GROUPS)),
        ],
        compiler_params=pltpu.CompilerParams(collective_id=0),
    )(A, B)
